# baseline (speedup 1.0000x reference)
import functools

import jax
import jax.numpy as jnp
from jax import lax
from jax.experimental import pallas as pl
from jax.experimental.pallas import tpu as pltpu
from jax.experimental.pallas import tpu_sc as plsc

_VOCAB = 100000
_D = 128
_B = 1024
_CTX = 20
_MAX_NORM = 1.0

_NC = 2
_NS = 16
_NW = _NC * _NS
_BPW = _B // _NW
_RPW = _BPW * _CTX
_GCH = 128
_NG = _RPW // _GCH
_LANES = 16
_DV = _D // _LANES


def _sc_embed_pool(x1d, table):
    mesh = plsc.VectorSubcoreMesh(core_axis_name="c", subcore_axis_name="s")

    @functools.partial(
        pl.kernel,
        mesh=mesh,
        out_type=jax.ShapeDtypeStruct((_B, _D), jnp.float32),
        scratch_types=[
            pltpu.VMEM((_RPW,), jnp.int32),
            pltpu.VMEM((_RPW, _D), jnp.float32),
            pltpu.VMEM((_BPW, _D), jnp.float32),
            pltpu.SemaphoreType.DMA,
        ],
    )
    def k(x_hbm, tab_hbm, h_hbm, idx_v, rows_v, h_v, sem):
        wid = lax.axis_index("s") * _NC + lax.axis_index("c")
        pltpu.sync_copy(x_hbm.at[pl.ds(wid * _RPW, _RPW)], idx_v)
        copies = [
            pltpu.async_copy(
                tab_hbm.at[idx_v.at[pl.ds(j * _GCH, _GCH)]],
                rows_v.at[pl.ds(j * _GCH, _GCH)],
                sem,
            )
            for j in range(_NG)
        ]
        for cp in copies:
            cp.wait()

        inv_ctx = jnp.float32(1.0 / _CTX)

        def per_item(i, carry):
            def per_ctx(c, accs):
                r = i * _CTX + c
                parts = [rows_v[r, pl.ds(_LANES * j, _LANES)] for j in range(_DV)]
                sq = parts[0] * parts[0]
                for p in parts[1:]:
                    sq = sq + p * p
                lanes = lax.iota(jnp.int32, _LANES)
                dnums = lax.GatherDimensionNumbers(
                    offset_dims=(), collapsed_slice_dims=(0,),
                    start_index_map=(0,))
                for step in (8, 4, 2, 1):
                    perm = lax.reshape(lanes ^ step, (_LANES, 1))
                    sq = sq + lax.gather(
                        sq, perm, dnums, (1,),
                        mode=lax.GatherScatterMode.PROMISE_IN_BOUNDS)
                xc = jnp.maximum(sq, 1.0)
                seed = jnp.full((_LANES,), 1.0, dtype=jnp.float32)
                for thr in (16.0, 256.0, 4096.0, 65536.0, 2.0**24):
                    seed = jnp.where(xc > thr, seed * 4.0, seed)
                nrm = seed
                for _ in range(5):
                    nrm = 0.5 * (nrm + xc / nrm)
                scale = jnp.where(sq > _MAX_NORM * _MAX_NORM,
                                  _MAX_NORM / (nrm + 1e-7), 1.0)
                return tuple(a + p * scale for a, p in zip(accs, parts))

            accs = lax.fori_loop(
                0, _CTX, per_ctx,
                tuple(jnp.zeros((_LANES,), jnp.float32) for _ in range(_DV)),
            )
            for j in range(_DV):
                h_v[i, pl.ds(_LANES * j, _LANES)] = accs[j] * inv_ctx
            return carry

        lax.fori_loop(0, _BPW, per_item, 0)
        pltpu.sync_copy(h_v, h_hbm.at[pl.ds(wid * _BPW, _BPW)])

    return k(x1d, table)


_TV = 512
_NBUF = 4


_COLS0 = (_VOCAB // _TV) * _TV
_TAIL = _VOCAB - _COLS0


def _tc_logits_main(h, W, b2):
    nsteps = _COLS0 // _TV

    def mm(h_ref, w_ref, b_ref, o_hbm, acc, sems):
        i = pl.program_id(0)
        k = lax.rem(i, _NBUF)

        @pl.when(i >= _NBUF)
        def _():
            pltpu.make_async_copy(
                acc.at[k],
                o_hbm.at[:, pl.ds((i - _NBUF) * _TV, _TV)],
                sems.at[k],
            ).wait()

        acc[k] = lax.dot_general(
            h_ref[...], w_ref[...], (((1,), (1,)), ((), ())),
            preferred_element_type=jnp.float32,
        ) + b_ref[...]

        pltpu.make_async_copy(
            acc.at[k], o_hbm.at[:, pl.ds(i * _TV, _TV)], sems.at[k]
        ).start()

        @pl.when(i == nsteps - 1)
        def _():
            for j in range(_NBUF):
                s = nsteps - _NBUF + j
                pltpu.make_async_copy(
                    acc.at[s % _NBUF],
                    o_hbm.at[:, pl.ds(s * _TV, _TV)],
                    sems.at[s % _NBUF],
                ).wait()

    return pl.pallas_call(
        mm,
        grid=(nsteps,),
        in_specs=[
            pl.BlockSpec((_B, _D), lambda i: (0, 0)),
            pl.BlockSpec((_TV, _D), lambda i: (i, 0)),
            pl.BlockSpec((1, _TV), lambda i: (0, i)),
        ],
        out_specs=pl.BlockSpec(memory_space=pl.ANY),
        out_shape=jax.ShapeDtypeStruct((_B, _VOCAB), jnp.float32),
        scratch_shapes=[
            pltpu.VMEM((_NBUF, _B, _TV), jnp.float32),
            pltpu.SemaphoreType.DMA((_NBUF,)),
        ],
    )(h, W, b2)


_TBLK = 256


def _tc_logits_tail(out, h, W, b2):
    blk = _COLS0 // _TBLK

    def mmt(o_in, h_ref, w_ref, b_ref, o_ref):
        del o_in
        o_ref[...] = lax.dot_general(
            h_ref[...], w_ref[...], (((1,), (1,)), ((), ())),
            preferred_element_type=jnp.float32,
        ) + b_ref[...]

    return pl.pallas_call(
        mmt,
        grid=(1,),
        in_specs=[
            pl.BlockSpec(memory_space=pl.ANY),
            pl.BlockSpec((_B, _D), lambda i: (0, 0)),
            pl.BlockSpec((_TBLK, _D), lambda i: (blk, 0)),
            pl.BlockSpec((1, _TBLK), lambda i: (0, blk)),
        ],
        out_specs=pl.BlockSpec((_B, _TBLK), lambda i: (0, blk)),
        out_shape=jax.ShapeDtypeStruct((_B, _VOCAB), jnp.float32),
        input_output_aliases={0: 0},
    )(out, h, W, b2)


def _tc_logits(h, W, b2):
    out = _tc_logits_main(h, W, b2)
    return _tc_logits_tail(out, h, W, b2)


def kernel(x, table, W, b):
    x1d = x.astype(jnp.int32).reshape(_B * _CTX)
    h = _sc_embed_pool(x1d, table)
    return _tc_logits(h, W, b.reshape(1, _VOCAB))


# speedup vs baseline: 1.0179x; 1.0179x over previous
import functools

import jax
import jax.numpy as jnp
from jax import lax
from jax.experimental import pallas as pl
from jax.experimental.pallas import tpu as pltpu
from jax.experimental.pallas import tpu_sc as plsc

_VOCAB = 100000
_D = 128
_B = 1024
_CTX = 20
_MAX_NORM = 1.0

_NC = 2
_NS = 16
_NW = _NC * _NS
_BPW = _B // _NW
_RPW = _BPW * _CTX
_GCH = 128
_NG = _RPW // _GCH
_LANES = 16
_DV = _D // _LANES


def _sc_embed_pool(x1d, table):
    mesh = plsc.VectorSubcoreMesh(core_axis_name="c", subcore_axis_name="s")

    @functools.partial(
        pl.kernel,
        mesh=mesh,
        out_type=jax.ShapeDtypeStruct((_B, _D), jnp.float32),
        scratch_types=[
            pltpu.VMEM((_RPW,), jnp.int32),
            pltpu.VMEM((_RPW, _D), jnp.float32),
            pltpu.VMEM((_BPW, _D), jnp.float32),
            pltpu.SemaphoreType.DMA,
        ],
    )
    def k(x_hbm, tab_hbm, h_hbm, idx_v, rows_v, h_v, sem):
        wid = lax.axis_index("s") * _NC + lax.axis_index("c")
        pltpu.sync_copy(x_hbm.at[pl.ds(wid * _RPW, _RPW)], idx_v)
        copies = [
            pltpu.async_copy(
                tab_hbm.at[idx_v.at[pl.ds(j * _GCH, _GCH)]],
                rows_v.at[pl.ds(j * _GCH, _GCH)],
                sem,
            )
            for j in range(_NG)
        ]
        for cp in copies:
            cp.wait()

        inv_ctx = jnp.float32(1.0 / _CTX)

        def per_item(i, carry):
            def per_ctx(c, accs):
                r = i * _CTX + c
                parts = [rows_v[r, pl.ds(_LANES * j, _LANES)] for j in range(_DV)]
                sq = parts[0] * parts[0]
                for p in parts[1:]:
                    sq = sq + p * p
                lanes = lax.iota(jnp.int32, _LANES)
                dnums = lax.GatherDimensionNumbers(
                    offset_dims=(), collapsed_slice_dims=(0,),
                    start_index_map=(0,))
                for step in (8, 4, 2, 1):
                    perm = lax.reshape(lanes ^ step, (_LANES, 1))
                    sq = sq + lax.gather(
                        sq, perm, dnums, (1,),
                        mode=lax.GatherScatterMode.PROMISE_IN_BOUNDS)
                xc = jnp.maximum(sq, 1.0)
                seed = jnp.full((_LANES,), 1.0, dtype=jnp.float32)
                for thr in (16.0, 256.0, 4096.0, 65536.0, 2.0**24):
                    seed = jnp.where(xc > thr, seed * 4.0, seed)
                nrm = seed
                for _ in range(5):
                    nrm = 0.5 * (nrm + xc / nrm)
                scale = jnp.where(sq > _MAX_NORM * _MAX_NORM,
                                  _MAX_NORM / (nrm + 1e-7), 1.0)
                return tuple(a + p * scale for a, p in zip(accs, parts))

            accs = lax.fori_loop(
                0, _CTX, per_ctx,
                tuple(jnp.zeros((_LANES,), jnp.float32) for _ in range(_DV)),
            )
            for j in range(_DV):
                h_v[i, pl.ds(_LANES * j, _LANES)] = accs[j] * inv_ctx
            return carry

        lax.fori_loop(0, _BPW, per_item, 0)
        pltpu.sync_copy(h_v, h_hbm.at[pl.ds(wid * _BPW, _BPW)])

    return k(x1d, table)


_TVC = 24960
_TB = 64
_COLS0 = (_VOCAB // _TVC) * _TVC
_TAIL = _VOCAB - _COLS0


def _tc_logits_main(h, W, b2):
    def mm(h_ref, w_ref, b_ref, o_ref):
        o_ref[...] = lax.dot_general(
            h_ref[...], w_ref[...], (((1,), (1,)), ((), ())),
            preferred_element_type=jnp.float32,
        ) + b_ref[...]

    return pl.pallas_call(
        mm,
        grid=(_COLS0 // _TVC, _B // _TB),
        in_specs=[
            pl.BlockSpec((_TB, _D), lambda c, b: (b, 0)),
            pl.BlockSpec((_TVC, _D), lambda c, b: (c, 0)),
            pl.BlockSpec((1, _TVC), lambda c, b: (0, c)),
        ],
        out_specs=pl.BlockSpec((_TB, _TVC), lambda c, b: (b, c)),
        out_shape=jax.ShapeDtypeStruct((_B, _VOCAB), jnp.float32),
    )(h, W, b2)


_TBLK = 256


def _tc_logits_tail(out, h, W, b2):
    blk = _COLS0 // _TBLK

    def mmt(o_in, h_ref, w_ref, b_ref, o_ref):
        del o_in
        o_ref[...] = lax.dot_general(
            h_ref[...], w_ref[...], (((1,), (1,)), ((), ())),
            preferred_element_type=jnp.float32,
        ) + b_ref[...]

    return pl.pallas_call(
        mmt,
        grid=(1,),
        in_specs=[
            pl.BlockSpec(memory_space=pl.ANY),
            pl.BlockSpec((_B, _D), lambda i: (0, 0)),
            pl.BlockSpec((_TBLK, _D), lambda i: (blk, 0)),
            pl.BlockSpec((1, _TBLK), lambda i: (0, blk)),
        ],
        out_specs=pl.BlockSpec((_B, _TBLK), lambda i: (0, blk)),
        out_shape=jax.ShapeDtypeStruct((_B, _VOCAB), jnp.float32),
        input_output_aliases={0: 0},
    )(out, h, W, b2)


def _tc_logits(h, W, b2):
    out = _tc_logits_main(h, W, b2)
    return _tc_logits_tail(out, h, W, b2)


def kernel(x, table, W, b):
    x1d = x.astype(jnp.int32).reshape(_B * _CTX)
    h = _sc_embed_pool(x1d, table)
    return _tc_logits(h, W, b.reshape(1, _VOCAB))


# speedup vs baseline: 2.7506x; 2.7023x over previous
import functools

import jax
import jax.numpy as jnp
from jax import lax
from jax.experimental import pallas as pl
from jax.experimental.pallas import tpu as pltpu
from jax.experimental.pallas import tpu_sc as plsc

_VOCAB = 100000
_D = 128
_B = 1024
_CTX = 20
_MAX_NORM = 1.0

_NC = 2
_NS = 16
_NW = _NC * _NS
_BPW = _B // _NW
_RPW = _BPW * _CTX
_GCH = 128
_NG = _RPW // _GCH
_LANES = 16
_DV = _D // _LANES


def _sc_embed_pool(x1d, table):
    mesh = plsc.VectorSubcoreMesh(core_axis_name="c", subcore_axis_name="s")

    @functools.partial(
        pl.kernel,
        mesh=mesh,
        out_type=jax.ShapeDtypeStruct((_B, _D), jnp.float32),
        scratch_types=[
            pltpu.VMEM((_RPW,), jnp.int32),
            pltpu.VMEM((_RPW, _D), jnp.float32),
            pltpu.VMEM((_BPW, _D), jnp.float32),
            pltpu.SemaphoreType.DMA,
        ],
    )
    def k(x_hbm, tab_hbm, h_hbm, idx_v, rows_v, h_v, sem):
        wid = lax.axis_index("s") * _NC + lax.axis_index("c")
        pltpu.sync_copy(x_hbm.at[pl.ds(wid * _RPW, _RPW)], idx_v)
        copies = [
            pltpu.async_copy(
                tab_hbm.at[idx_v.at[pl.ds(j * _GCH, _GCH)]],
                rows_v.at[pl.ds(j * _GCH, _GCH)],
                sem,
            )
            for j in range(_NG)
        ]
        for cp in copies:
            cp.wait()

        inv_ctx = jnp.float32(1.0 / _CTX)

        def per_item(i, carry):
            def per_ctx(c, accs):
                r = i * _CTX + c
                parts = [rows_v[r, pl.ds(_LANES * j, _LANES)] for j in range(_DV)]
                sq = parts[0] * parts[0]
                for p in parts[1:]:
                    sq = sq + p * p
                lanes = lax.iota(jnp.int32, _LANES)
                dnums = lax.GatherDimensionNumbers(
                    offset_dims=(), collapsed_slice_dims=(0,),
                    start_index_map=(0,))
                for step in (8, 4, 2, 1):
                    perm = lax.reshape(lanes ^ step, (_LANES, 1))
                    sq = sq + lax.gather(
                        sq, perm, dnums, (1,),
                        mode=lax.GatherScatterMode.PROMISE_IN_BOUNDS)
                xc = jnp.maximum(sq, 1.0)
                seed = jnp.full((_LANES,), 1.0, dtype=jnp.float32)
                for thr in (16.0, 256.0, 4096.0, 65536.0, 2.0**24):
                    seed = jnp.where(xc > thr, seed * 4.0, seed)
                nrm = seed
                for _ in range(5):
                    nrm = 0.5 * (nrm + xc / nrm)
                scale = jnp.where(sq > _MAX_NORM * _MAX_NORM,
                                  _MAX_NORM / (nrm + 1e-7), 1.0)
                return tuple(a + p * scale for a, p in zip(accs, parts))

            accs = lax.fori_loop(
                0, _CTX, per_ctx,
                tuple(jnp.zeros((_LANES,), jnp.float32) for _ in range(_DV)),
            )
            for j in range(_DV):
                h_v[i, pl.ds(_LANES * j, _LANES)] = accs[j] * inv_ctx
            return carry

        lax.fori_loop(0, _BPW, per_item, 0)
        pltpu.sync_copy(h_v, h_hbm.at[pl.ds(wid * _BPW, _BPW)])

    return k(x1d, table)


_TV = 4096


def _tc_logits_t(h, W, bcol):
    def mm(w_ref, h_ref, b_ref, o_ref):
        o_ref[...] = lax.dot_general(
            w_ref[...], h_ref[...], (((1,), (1,)), ((), ())),
            preferred_element_type=jnp.float32,
        ) + b_ref[...]

    return pl.pallas_call(
        mm,
        grid=(pl.cdiv(_VOCAB, _TV),),
        in_specs=[
            pl.BlockSpec((_TV, _D), lambda c: (c, 0)),
            pl.BlockSpec((_B, _D), lambda c: (0, 0)),
            pl.BlockSpec((_TV, 1), lambda c: (c, 0)),
        ],
        out_specs=pl.BlockSpec((_TV, _B), lambda c: (c, 0)),
        out_shape=jax.ShapeDtypeStruct((_VOCAB, _B), jnp.float32),
    )(W, h, bcol)


def kernel(x, table, W, b):
    x1d = x.astype(jnp.int32).reshape(_B * _CTX)
    h = _sc_embed_pool(x1d, table)
    logits_t = _tc_logits_t(h, W, b.reshape(_VOCAB, 1))
    return logits_t.T
